# baseline (speedup 1.0000x reference)
import functools

import jax
import jax.numpy as jnp
from jax import lax
from jax.experimental import pallas as pl
from jax.experimental.pallas import tpu as pltpu
from jax.experimental.pallas import tpu_sc as plsc

_EW = 20


def _make_emb_gather(B, d0, d1, d2, v0, v1, v2):
    info = plsc.get_sparse_core_info()
    NC, NS = info.num_cores, info.num_subcores
    NW = NC * NS
    bpw = B // NW
    ngrp = bpw // 16

    mesh = plsc.VectorSubcoreMesh(core_axis_name="c", subcore_axis_name="s")

    @functools.partial(
        pl.kernel,
        mesh=mesh,
        out_type=jax.ShapeDtypeStruct((B, _EW), jnp.float32),
        compiler_params=pltpu.CompilerParams(use_tc_tiling_on_sc=False,
                                             needs_layout_passes=False),
        scratch_types=[
            pltpu.VMEM((3, bpw), jnp.int32),
            pltpu.VMEM((d0, v0), jnp.float32),
            pltpu.VMEM((d1, v1), jnp.float32),
            pltpu.VMEM((d2, v2), jnp.float32),
            pltpu.VMEM((bpw, _EW), jnp.float32),
            pltpu.SemaphoreType.DMA,
        ],
    )
    def emb_gather(cat_hbm, t0_hbm, t1_hbm, t2_hbm, out_hbm,
                   cat_v, t0_v, t1_v, t2_v, out_v, sem):
        wid = lax.axis_index("s") * NC + lax.axis_index("c")
        base = wid * bpw
        cps = [pltpu.async_copy(cat_hbm.at[:, pl.ds(base, bpw)], cat_v, sem),
               pltpu.async_copy(t0_hbm, t0_v, sem),
               pltpu.async_copy(t1_hbm, t1_v, sem),
               pltpu.async_copy(t2_hbm, t2_v, sem)]
        for c in cps:
            c.wait()

        lanes = lax.iota(jnp.int32, 16)

        def body(g, carry):
            rows = g * 16 + lanes
            col = lambda c: jnp.full((16,), c, jnp.int32)
            iv0 = plsc.load_gather(cat_v, [col(0), rows])
            iv1 = plsc.load_gather(cat_v, [col(1), rows])
            iv2 = plsc.load_gather(cat_v, [col(2), rows])
            for l in range(d0):
                v = plsc.load_gather(t0_v, [col(l), iv0])
                plsc.store_scatter(out_v, [rows, col(l)], v)
            for l in range(d1):
                v = plsc.load_gather(t1_v, [col(l), iv1])
                plsc.store_scatter(out_v, [rows, col(d0 + l)], v)
            for l in range(d2):
                v = plsc.load_gather(t2_v, [col(l), iv2])
                plsc.store_scatter(out_v, [rows, col(d0 + d1 + l)], v)
            return carry

        lax.fori_loop(0, ngrp, body, 0)
        pltpu.sync_copy(out_v, out_hbm.at[pl.ds(base, bpw)])

    return emb_gather


def _dot(x, w):
    return lax.dot_general(x, w, (((1,), (0,)), ((), ())),
                           preferred_element_type=jnp.float32)


def _dotnt(x, w):
    return lax.dot_general(x, w, (((1,), (1,)), ((), ())),
                           preferred_element_type=jnp.float32)


def _mlp_body(contt_ref, xe_ref, w0et_ref, w0ct_ref, b0_ref, w1t_ref,
              b1_ref, w2_ref, b2_ref, w3_ref, b3_ref, w4_ref, b4_ref,
              out_ref):
    x0 = (_dot(xe_ref[...], w0et_ref[...])
          + lax.dot_general(contt_ref[...], w0ct_ref[...],
                            (((0,), (0,)), ((), ())),
                            preferred_element_type=jnp.float32)
          + b0_ref[...][None, :])
    h = jnp.maximum(x0, 0.0)
    h = jnp.maximum(_dot(h, w1t_ref[...]) + b1_ref[...][None, :], 0.0)
    h = jnp.maximum(_dotnt(h, w2_ref[...]) + b2_ref[...][None, :], 0.0)
    h = jnp.maximum(_dotnt(h, w3_ref[...]) + b3_ref[...][None, :], 0.0)
    logits_t = (lax.dot_general(w4_ref[...], h, (((1,), (1,)), ((), ())),
                                preferred_element_type=jnp.float32)
                + b4_ref[...][:, None])
    out_ref[...] = jax.nn.log_softmax(logits_t, axis=0)


def _run_mlp(contt, xe, w0et, w0ct, b0, w1t, b1, w2, b2, w3, b3, w4, b4,
             blk):
    B = contt.shape[1]
    grid = (B // blk,)
    row = lambda i: (i, 0)
    colb = lambda i: (0, i)
    whole2 = lambda i: (0, 0)
    whole1 = lambda i: (0,)
    wspec = lambda w: pl.BlockSpec(w.shape, whole2)
    bspec = lambda b: pl.BlockSpec(b.shape, whole1)
    outt = pl.pallas_call(
        _mlp_body,
        grid=grid,
        in_specs=[
            pl.BlockSpec((contt.shape[0], blk), colb),
            pl.BlockSpec((blk, _EW), row),
            wspec(w0et), wspec(w0ct), bspec(b0), wspec(w1t), bspec(b1),
            wspec(w2), bspec(b2), wspec(w3), bspec(b3), wspec(w4),
            bspec(b4),
        ],
        out_specs=pl.BlockSpec((2, blk), colb),
        out_shape=jax.ShapeDtypeStruct((2, B), jnp.float32),
    )(contt, xe, w0et, w0ct, b0, w1t, b1, w2, b2, w3, b3, w4, b4)
    return outt.T


def kernel(cont_data, cat_data, E0, E1, E2, W0, b0, W1, b1, W2, b2, W3, b3,
           W4, b4):
    B = cont_data.shape[0]
    emb_gather = _make_emb_gather(B, E0.shape[1], E1.shape[1], E2.shape[1],
                                  E0.shape[0], E1.shape[0], E2.shape[0])
    xe = emb_gather(cat_data.astype(jnp.int32).T, E0.T, E1.T, E2.T)
    w0t = W0.T
    return _run_mlp(cont_data.T, xe, w0t[:_EW], w0t[_EW:], b0, W1.T, b1,
                    W2, b2, W3, b3, W4, b4, blk=4096)


# speedup vs baseline: 1.0058x; 1.0058x over previous
import functools

import jax
import jax.numpy as jnp
from jax import lax
from jax.experimental import pallas as pl
from jax.experimental.pallas import tpu as pltpu
from jax.experimental.pallas import tpu_sc as plsc

_EW = 20


def _make_emb_gather(B, d0, d1, d2, v0, v1, v2):
    info = plsc.get_sparse_core_info()
    NC, NS = info.num_cores, info.num_subcores
    NW = NC * NS
    bpw = B // NW
    ngrp = bpw // 16

    mesh = plsc.VectorSubcoreMesh(core_axis_name="c", subcore_axis_name="s")

    @functools.partial(
        pl.kernel,
        mesh=mesh,
        out_type=jax.ShapeDtypeStruct((B, _EW), jnp.float32),
        compiler_params=pltpu.CompilerParams(use_tc_tiling_on_sc=False,
                                             needs_layout_passes=False),
        scratch_types=[
            pltpu.VMEM((3, bpw), jnp.int32),
            pltpu.VMEM((d0, v0), jnp.float32),
            pltpu.VMEM((d1, v1), jnp.float32),
            pltpu.VMEM((d2, v2), jnp.float32),
            pltpu.VMEM((bpw, _EW), jnp.float32),
            pltpu.SemaphoreType.DMA,
        ],
    )
    def emb_gather(cat_hbm, t0_hbm, t1_hbm, t2_hbm, out_hbm,
                   cat_v, t0_v, t1_v, t2_v, out_v, sem):
        wid = lax.axis_index("s") * NC + lax.axis_index("c")
        base = wid * bpw
        cps = [pltpu.async_copy(cat_hbm.at[:, pl.ds(base, bpw)], cat_v, sem),
               pltpu.async_copy(t0_hbm, t0_v, sem),
               pltpu.async_copy(t1_hbm, t1_v, sem),
               pltpu.async_copy(t2_hbm, t2_v, sem)]
        for c in cps:
            c.wait()

        lanes = lax.iota(jnp.int32, 16)

        def body(g, carry):
            rows = g * 16 + lanes
            col = lambda c: jnp.full((16,), c, jnp.int32)
            iv0 = plsc.load_gather(cat_v, [col(0), rows])
            iv1 = plsc.load_gather(cat_v, [col(1), rows])
            iv2 = plsc.load_gather(cat_v, [col(2), rows])
            for l in range(d0):
                v = plsc.load_gather(t0_v, [col(l), iv0])
                plsc.store_scatter(out_v, [rows, col(l)], v)
            for l in range(d1):
                v = plsc.load_gather(t1_v, [col(l), iv1])
                plsc.store_scatter(out_v, [rows, col(d0 + l)], v)
            for l in range(d2):
                v = plsc.load_gather(t2_v, [col(l), iv2])
                plsc.store_scatter(out_v, [rows, col(d0 + d1 + l)], v)
            return carry

        lax.fori_loop(0, ngrp, body, 0, unroll=4)
        pltpu.sync_copy(out_v, out_hbm.at[pl.ds(base, bpw)])

    return emb_gather


def _dot(x, w):
    return lax.dot_general(x, w, (((1,), (0,)), ((), ())),
                           preferred_element_type=jnp.float32)


def _dotnt(x, w):
    return lax.dot_general(x, w, (((1,), (1,)), ((), ())),
                           preferred_element_type=jnp.float32)


def _mlp_body(contt_ref, xe_ref, w0et_ref, w0ct_ref, b0_ref, w1t_ref,
              b1_ref, w2_ref, b2_ref, w3_ref, b3_ref, w4_ref, b4_ref,
              out_ref):
    x0 = (_dot(xe_ref[...], w0et_ref[...])
          + lax.dot_general(contt_ref[...], w0ct_ref[...],
                            (((0,), (0,)), ((), ())),
                            preferred_element_type=jnp.float32)
          + b0_ref[...][None, :])
    h = jnp.maximum(x0, 0.0)
    h = jnp.maximum(_dot(h, w1t_ref[...]) + b1_ref[...][None, :], 0.0)
    h = jnp.maximum(_dotnt(h, w2_ref[...]) + b2_ref[...][None, :], 0.0)
    h = jnp.maximum(_dotnt(h, w3_ref[...]) + b3_ref[...][None, :], 0.0)
    logits_t = (lax.dot_general(w4_ref[...], h, (((1,), (1,)), ((), ())),
                                preferred_element_type=jnp.float32)
                + b4_ref[...][:, None])
    out_ref[...] = jax.nn.log_softmax(logits_t, axis=0)


def _run_mlp(contt, xe, w0et, w0ct, b0, w1t, b1, w2, b2, w3, b3, w4, b4,
             blk):
    B = contt.shape[1]
    grid = (B // blk,)
    row = lambda i: (i, 0)
    colb = lambda i: (0, i)
    whole2 = lambda i: (0, 0)
    whole1 = lambda i: (0,)
    wspec = lambda w: pl.BlockSpec(w.shape, whole2)
    bspec = lambda b: pl.BlockSpec(b.shape, whole1)
    outt = pl.pallas_call(
        _mlp_body,
        grid=grid,
        in_specs=[
            pl.BlockSpec((contt.shape[0], blk), colb),
            pl.BlockSpec((blk, _EW), row),
            wspec(w0et), wspec(w0ct), bspec(b0), wspec(w1t), bspec(b1),
            wspec(w2), bspec(b2), wspec(w3), bspec(b3), wspec(w4),
            bspec(b4),
        ],
        out_specs=pl.BlockSpec((2, blk), colb),
        out_shape=jax.ShapeDtypeStruct((2, B), jnp.float32),
    )(contt, xe, w0et, w0ct, b0, w1t, b1, w2, b2, w3, b3, w4, b4)
    return outt.T


def kernel(cont_data, cat_data, E0, E1, E2, W0, b0, W1, b1, W2, b2, W3, b3,
           W4, b4):
    B = cont_data.shape[0]
    emb_gather = _make_emb_gather(B, E0.shape[1], E1.shape[1], E2.shape[1],
                                  E0.shape[0], E1.shape[0], E2.shape[0])
    xe = emb_gather(cat_data.astype(jnp.int32).T, E0.T, E1.T, E2.T)
    w0t = W0.T
    return _run_mlp(cont_data.T, xe, w0t[:_EW], w0t[_EW:], b0, W1.T, b1,
                    W2, b2, W3, b3, W4, b4, blk=4096)
